# baseline (speedup 1.0000x reference)
import functools

import jax
import jax.numpy as jnp
from jax import lax
from jax.experimental import pallas as pl
from jax.experimental.pallas import tpu as pltpu
from jax.experimental.pallas import tpu_sc as plsc

NPTS = 262144
D = 16
NBUCKETS = 4194304
MASK = NBUCKETS - 1
P0 = 1
P1 = -1640531535
P2 = 805459861

NWORKERS = 32
PPW = NPTS // NWORKERS
C = 256
NCHUNK = PPW // C
E = 8 * C
NSTREAM = E // 128

_mesh = plsc.VectorSubcoreMesh(core_axis_name="c", subcore_axis_name="s")


@functools.partial(
    pl.kernel,
    out_type=jax.ShapeDtypeStruct((NPTS, D), jnp.float32),
    mesh=_mesh,
    scratch_types=[
        pltpu.VMEM((C,), jnp.float32),
        pltpu.VMEM((C,), jnp.float32),
        pltpu.VMEM((C,), jnp.float32),
        pltpu.VMEM((E,), jnp.int32),
        pltpu.VMEM((E,), jnp.float32),
        pltpu.VMEM((E,), jnp.int32),
        pltpu.VMEM((E, D), jnp.float32),
        pltpu.VMEM((C, D), jnp.float32),
        pltpu.SemaphoreType.DMA,
        pltpu.SemaphoreType.DMA,
    ],
    compiler_params=pltpu.CompilerParams(
        needs_layout_passes=False, use_tc_tiling_on_sc=False),
)
def _voxel_sc(xs_hbm, ys_hbm, zs_hbm, h2v_hbm, vf_hbm, out_hbm,
              xs_v, ys_v, zs_v, hv_v, w_v, vid_v, feats_v, out_v,
              sem_h, sem_f):
    i32 = jnp.int32
    wid = lax.axis_index("s") * i32(2) + lax.axis_index("c")
    base_pt = wid * i32(PPW)
    iota = lax.broadcasted_iota(jnp.int32, (16,), 0)

    def chunk_body(ci, _):
        off = base_pt + ci * i32(C)
        pltpu.sync_copy(xs_hbm.at[pl.ds(off, C)], xs_v)
        pltpu.sync_copy(ys_hbm.at[pl.ds(off, C)], ys_v)
        pltpu.sync_copy(zs_hbm.at[pl.ds(off, C)], zs_v)

        def hash_body(l, _):
            lb = l * i32(16)
            x = xs_v[pl.ds(lb, 16)]
            y = ys_v[pl.ds(lb, 16)]
            z = zs_v[pl.ds(lb, 16)]
            qx = (x + 1.0) * 64.0
            qy = (y + 1.0) * 64.0
            qz = (z + 1.0) * 64.0
            bx = qx.astype(jnp.int32)
            by = qy.astype(jnp.int32)
            bz = qz.astype(jnp.int32)
            fx = qx - bx.astype(jnp.float32)
            fy = qy - by.astype(jnp.float32)
            fz = qz - bz.astype(jnp.float32)
            s = bx * jnp.int32(P0) + by * jnp.int32(P1) + bz * jnp.int32(P2)
            wx = (1.0 - fx, fx)
            wy = (1.0 - fy, fy)
            wz = (1.0 - fz, fz)
            base_idx = iota * i32(8) + l * i32(128)
            for c in range(8):
                cx, cy, cz = c & 1, (c >> 1) & 1, (c >> 2) & 1
                dc = (cx * 1 + cy * 2654435761 + cz * 805459861) % (1 << 32)
                if dc >= 1 << 31:
                    dc -= 1 << 32
                hv = ((s + jnp.int32(dc)) & jnp.int32(MASK)) * i32(2)
                w = wx[cx] * wy[cy] * wz[cz]
                idxv = base_idx + i32(c)
                plsc.store_scatter(hv_v, [idxv], hv)
                plsc.store_scatter(w_v, [idxv], w)

        lax.fori_loop(jnp.int32(0), jnp.int32(C // 16), hash_body, None)

        copies = []
        for j in range(NSTREAM):
            copies.append(pltpu.async_copy(
                h2v_hbm.at[hv_v.at[pl.ds(j * 128, 128)]],
                vid_v.at[pl.ds(j * 128, 128)],
                sem_h))
        for cp in copies:
            cp.wait()

        def clamp_body(k, _):
            kb = k * i32(16)
            lo = vid_v[pl.ds(kb, 16)]
            valid = lo >= 0
            hv_v[pl.ds(kb, 16)] = jnp.maximum(lo, i32(0))
            w = w_v[pl.ds(kb, 16)]
            w_v[pl.ds(kb, 16)] = jnp.where(valid, w, 0.0)

        lax.fori_loop(jnp.int32(0), jnp.int32(E // 16), clamp_body, None)

        copies = []
        for j in range(NSTREAM):
            copies.append(pltpu.async_copy(
                vf_hbm.at[hv_v.at[pl.ds(j * 128, 128)]],
                feats_v.at[pl.ds(j * 128, 128)],
                sem_f))
        for cp in copies:
            cp.wait()

        def blend_body(p2, _):
            b = p2 * i32(16)
            w16 = w_v[pl.ds(b, 16)]
            acc0 = w16[0] * feats_v[b, :]
            acc1 = w16[8] * feats_v[b + i32(8), :]
            for c in range(1, 8):
                acc0 = acc0 + w16[c] * feats_v[b + i32(c), :]
                acc1 = acc1 + w16[8 + c] * feats_v[b + i32(8 + c), :]
            out_v[p2 * i32(2), :] = acc0
            out_v[p2 * i32(2) + i32(1), :] = acc1

        lax.fori_loop(jnp.int32(0), jnp.int32(C // 2), blend_body, None)

        pltpu.sync_copy(out_v, out_hbm.at[pl.ds(off, C)])

    lax.fori_loop(jnp.int32(0), jnp.int32(NCHUNK), chunk_body, None)


def kernel(pts, voxel_features, hash2vox, primes):
    del primes
    pts = pts.astype(jnp.float32)
    xs = pts[:, 0]
    ys = pts[:, 1]
    zs = pts[:, 2]
    h2v = lax.bitcast_convert_type(hash2vox.astype(jnp.int64), jnp.int32)
    h2v = h2v.reshape(-1)
    vf = voxel_features.astype(jnp.float32)
    return _voxel_sc(xs, ys, zs, h2v, vf)


# speedup vs baseline: 5.1177x; 5.1177x over previous
import functools

import jax
import jax.numpy as jnp
from jax import lax
from jax.experimental import pallas as pl
from jax.experimental.pallas import tpu as pltpu
from jax.experimental.pallas import tpu_sc as plsc

NPTS = 262144
D = 16
NBUCKETS = 4194304
MASK = NBUCKETS - 1
P0 = 1
P1 = -1640531535
P2 = 805459861

NWORKERS = 32
PPW = NPTS // NWORKERS
C = 256
NCHUNK = PPW // C
E = 8 * C
NSTREAM = E // 128

_mesh = plsc.VectorSubcoreMesh(core_axis_name="c", subcore_axis_name="s")


@functools.partial(
    pl.kernel,
    out_type=jax.ShapeDtypeStruct((NPTS, D), jnp.float32),
    mesh=_mesh,
    scratch_types=[
        pltpu.VMEM((C,), jnp.float32),
        pltpu.VMEM((C,), jnp.float32),
        pltpu.VMEM((C,), jnp.float32),
        pltpu.VMEM((E,), jnp.int32),
        pltpu.VMEM((E,), jnp.float32),
        pltpu.VMEM((E,), jnp.int32),
        pltpu.VMEM((E, D), jnp.float32),
        pltpu.VMEM((C, D), jnp.float32),
        pltpu.SemaphoreType.DMA,
        pltpu.SemaphoreType.DMA,
    ],
    compiler_params=pltpu.CompilerParams(
        needs_layout_passes=False, use_tc_tiling_on_sc=False),
)
def _voxel_sc(xs_hbm, ys_hbm, zs_hbm, h2v_hbm, vf_hbm, out_hbm,
              xs_v, ys_v, zs_v, hv_v, w_v, vid_v, feats_v, out_v,
              sem_h, sem_f):
    i32 = jnp.int32
    wid = lax.axis_index("s") * i32(2) + lax.axis_index("c")
    base_pt = wid * i32(PPW)
    iota = lax.broadcasted_iota(jnp.int32, (16,), 0)

    def chunk_body(ci, _):
        off = base_pt + ci * i32(C)
        pltpu.sync_copy(xs_hbm.at[pl.ds(off, C)], xs_v)
        pltpu.sync_copy(ys_hbm.at[pl.ds(off, C)], ys_v)
        pltpu.sync_copy(zs_hbm.at[pl.ds(off, C)], zs_v)

        def hash_body(l, _):
            lb = l * i32(16)
            x = xs_v[pl.ds(lb, 16)]
            y = ys_v[pl.ds(lb, 16)]
            z = zs_v[pl.ds(lb, 16)]
            qx = (x + 1.0) * 64.0
            qy = (y + 1.0) * 64.0
            qz = (z + 1.0) * 64.0
            bx = qx.astype(jnp.int32)
            by = qy.astype(jnp.int32)
            bz = qz.astype(jnp.int32)
            fx = qx - bx.astype(jnp.float32)
            fy = qy - by.astype(jnp.float32)
            fz = qz - bz.astype(jnp.float32)
            s = bx * jnp.int32(P0) + by * jnp.int32(P1) + bz * jnp.int32(P2)
            wx = (1.0 - fx, fx)
            wy = (1.0 - fy, fy)
            wz = (1.0 - fz, fz)
            base_idx = iota * i32(8) + l * i32(128)
            for c in range(8):
                cx, cy, cz = c & 1, (c >> 1) & 1, (c >> 2) & 1
                dc = (cx * 1 + cy * 2654435761 + cz * 805459861) % (1 << 32)
                if dc >= 1 << 31:
                    dc -= 1 << 32
                hv = (s + jnp.int32(dc)) & jnp.int32(MASK)
                w = wx[cx] * wy[cy] * wz[cz]
                idxv = base_idx + i32(c)
                plsc.store_scatter(hv_v, [idxv], hv)
                plsc.store_scatter(w_v, [idxv], w)

        lax.fori_loop(jnp.int32(0), jnp.int32(C // 16), hash_body, None)

        copies = []
        for j in range(NSTREAM):
            copies.append(pltpu.async_copy(
                h2v_hbm.at[hv_v.at[pl.ds(j * 128, 128)]],
                vid_v.at[pl.ds(j * 128, 128)],
                sem_h))
        for cp in copies:
            cp.wait()

        def clamp_body(k, _):
            kb = k * i32(16)
            lo = vid_v[pl.ds(kb, 16)]
            valid = lo >= 0
            hv_v[pl.ds(kb, 16)] = jnp.maximum(lo, i32(0))
            w = w_v[pl.ds(kb, 16)]
            w_v[pl.ds(kb, 16)] = jnp.where(valid, w, 0.0)

        lax.fori_loop(jnp.int32(0), jnp.int32(E // 16), clamp_body, None)

        copies = []
        for j in range(NSTREAM):
            copies.append(pltpu.async_copy(
                vf_hbm.at[hv_v.at[pl.ds(j * 128, 128)]],
                feats_v.at[pl.ds(j * 128, 128)],
                sem_f))
        for cp in copies:
            cp.wait()

        def blend_body(p2, _):
            b = p2 * i32(16)
            w16 = w_v[pl.ds(b, 16)]
            acc0 = w16[0] * feats_v[b, :]
            acc1 = w16[8] * feats_v[b + i32(8), :]
            for c in range(1, 8):
                acc0 = acc0 + w16[c] * feats_v[b + i32(c), :]
                acc1 = acc1 + w16[8 + c] * feats_v[b + i32(8 + c), :]
            out_v[p2 * i32(2), :] = acc0
            out_v[p2 * i32(2) + i32(1), :] = acc1

        lax.fori_loop(jnp.int32(0), jnp.int32(C // 2), blend_body, None)

        pltpu.sync_copy(out_v, out_hbm.at[pl.ds(off, C)])

    lax.fori_loop(jnp.int32(0), jnp.int32(NCHUNK), chunk_body, None)


def kernel(pts, voxel_features, hash2vox, primes):
    del primes
    pts = pts.astype(jnp.float32)
    xs = pts[:, 0]
    ys = pts[:, 1]
    zs = pts[:, 2]
    h2v = hash2vox.astype(jnp.int32)
    vf = voxel_features.astype(jnp.float32)
    return _voxel_sc(xs, ys, zs, h2v, vf)
